# baseline (speedup 1.0000x reference)
import jax
import jax.numpy as jnp
from jax.experimental import pallas as pl

N_CLS = 1000
N_CTX = 16
DIM = 768
SUF = 60
SEQ = 77
C_BLK = 8


def _body(ctx_ref, ctx_neg_ref, pre_ref, pre_neg_ref, suf_ref, suf_neg_ref,
          out_ref, out_neg_ref):
    out_ref[:, 0:1, :] = pre_ref[...]
    out_ref[:, 1:1 + N_CTX, :] = jnp.broadcast_to(
        ctx_ref[...][None, :, :], (C_BLK, N_CTX, DIM))
    out_ref[:, 1 + N_CTX:, :] = suf_ref[...]
    out_neg_ref[:, 0:1, :] = pre_neg_ref[...]
    out_neg_ref[:, 1:1 + N_CTX, :] = jnp.broadcast_to(
        ctx_neg_ref[...][None, :, :], (C_BLK, N_CTX, DIM))
    out_neg_ref[:, 1 + N_CTX:, :] = suf_neg_ref[...]


def kernel(ctx, ctx_neg, token_prefix, token_prefix_neg, token_suffix,
           token_suffix_neg):
    n_cls = token_prefix.shape[0]
    grid = (n_cls // C_BLK,)
    out_shape = jax.ShapeDtypeStruct((n_cls, SEQ, DIM), jnp.float32)
    prompts, prompts_neg = pl.pallas_call(
        _body,
        grid=grid,
        in_specs=[
            pl.BlockSpec((N_CTX, DIM), lambda i: (0, 0)),
            pl.BlockSpec((N_CTX, DIM), lambda i: (0, 0)),
            pl.BlockSpec((C_BLK, 1, DIM), lambda i: (i, 0, 0)),
            pl.BlockSpec((C_BLK, 1, DIM), lambda i: (i, 0, 0)),
            pl.BlockSpec((C_BLK, SUF, DIM), lambda i: (i, 0, 0)),
            pl.BlockSpec((C_BLK, SUF, DIM), lambda i: (i, 0, 0)),
        ],
        out_specs=[
            pl.BlockSpec((C_BLK, SEQ, DIM), lambda i: (i, 0, 0)),
            pl.BlockSpec((C_BLK, SEQ, DIM), lambda i: (i, 0, 0)),
        ],
        out_shape=[out_shape, out_shape],
    )(ctx, ctx_neg, token_prefix, token_prefix_neg, token_suffix,
      token_suffix_neg)
    return (prompts, prompts_neg)
